# baseline (device time: 25761 ns/iter reference)
import jax
import jax.numpy as jnp
from jax import lax
from jax.experimental import pallas as pl
from jax.experimental.pallas import tpu as pltpu

N_DEV = 8
NC = 4


def kernel(x, W, labels):
    T, D = x.shape
    _, V = W.shape
    C = V // NC

    def body(x_ref, w_ref, labels_ref, out_ref,
             sacc_ref, comm_ref, send_sems, recv_sems):
        j = pl.program_id(0)
        my_pos = lax.axis_index("i")

        x_bf = x_ref[...].astype(jnp.bfloat16)
        w_bf = w_ref[...].astype(jnp.bfloat16)
        logits = lax.dot_general(
            x_bf, w_bf, (((1,), (0,)), ((), ())),
            preferred_element_type=jnp.float32,
        )
        s_chunk = jnp.sum(jnp.exp(logits), axis=1)

        lab = labels_ref[0, :]
        local_col = lab - my_pos * V - j * C
        col = lax.broadcasted_iota(jnp.int32, (T, C), 1)
        hit = col == local_col[:, None]
        c_chunk = jnp.sum(jnp.where(hit, logits, 0.0), axis=1)

        st = jnp.stack([s_chunk, c_chunk])[None]

        @pl.when(j == 0)
        def _():
            sacc_ref[...] = st

        @pl.when(j > 0)
        def _():
            sacc_ref[...] = sacc_ref[...] + st

        @pl.when(j == NC - 1)
        def _():
            barrier_sem = pltpu.get_barrier_semaphore()
            for d in range(1, N_DEV):
                peer = (my_pos + d) % N_DEV
                pl.semaphore_signal(
                    barrier_sem, inc=1,
                    device_id=(peer,), device_id_type=pl.DeviceIdType.MESH,
                )
            pl.semaphore_wait(barrier_sem, N_DEV - 1)

            sends = []
            for d in range(1, N_DEV):
                peer = (my_pos + d) % N_DEV
                rdma = pltpu.make_async_remote_copy(
                    src_ref=sacc_ref,
                    dst_ref=comm_ref.at[pl.ds(my_pos, 1)],
                    send_sem=send_sems.at[d],
                    recv_sem=recv_sems.at[my_pos],
                    device_id=(peer,),
                    device_id_type=pl.DeviceIdType.MESH,
                )
                rdma.start()
                sends.append(rdma)

            for d in range(1, N_DEV):
                src_pos = (my_pos + d) % N_DEV
                recv = pltpu.make_async_remote_copy(
                    src_ref=sacc_ref,
                    dst_ref=comm_ref.at[pl.ds(src_pos, 1)],
                    send_sem=send_sems.at[d],
                    recv_sem=recv_sems.at[src_pos],
                    device_id=(src_pos,),
                    device_id_type=pl.DeviceIdType.MESH,
                )
                recv.wait_recv()
            for rdma in sends:
                rdma.wait_send()

            all_st = comm_ref[...]
            dev = lax.broadcasted_iota(jnp.int32, (N_DEV, 2, T), 0)
            all_st = jnp.where(dev == my_pos, sacc_ref[...], all_st)
            S = jnp.sum(all_st[:, 0, :], axis=0)
            L = jnp.sum(all_st[:, 1, :], axis=0)
            out_ref[0, :] = jnp.log(S) - L

    out = pl.pallas_call(
        body,
        grid=(NC,),
        out_shape=jax.ShapeDtypeStruct((1, T), jnp.float32),
        in_specs=[
            pl.BlockSpec((T, D), lambda j: (0, 0)),
            pl.BlockSpec((D, C), lambda j: (0, j)),
            pl.BlockSpec((1, T), lambda j: (0, 0)),
        ],
        out_specs=pl.BlockSpec((1, T), lambda j: (0, 0)),
        scratch_shapes=[
            pltpu.VMEM((1, 2, T), jnp.float32),
            pltpu.VMEM((N_DEV, 2, T), jnp.float32),
            pltpu.SemaphoreType.DMA((N_DEV,)),
            pltpu.SemaphoreType.DMA((N_DEV,)),
        ],
        compiler_params=pltpu.CompilerParams(
            collective_id=0,
            dimension_semantics=("arbitrary",),
        ),
    )(x, W, labels.reshape(1, T))
    return out.reshape(T)


# device time: 25200 ns/iter; 1.0223x vs baseline; 1.0223x over previous
import jax
import jax.numpy as jnp
from jax import lax
from jax.experimental import pallas as pl
from jax.experimental.pallas import tpu as pltpu

N_DEV = 8
NC = 8
S = 4


def kernel(x, W, labels):
    T, D = x.shape
    _, V = W.shape
    C = V // NC
    R = D // S

    def body(x_ref, w_hbm, labels_ref, out_ref,
             wbuf, sacc_ref, comm_ref, dma_sems, send_sems, recv_sems):
        my_pos = lax.axis_index("i")

        barrier_sem = pltpu.get_barrier_semaphore()
        for d in range(1, N_DEV):
            peer = (my_pos + d) % N_DEV
            pl.semaphore_signal(
                barrier_sem, inc=1,
                device_id=(peer,), device_id_type=pl.DeviceIdType.MESH,
            )

        def chunk_copy(j, si):
            return pltpu.make_async_copy(
                w_hbm.at[pl.ds(si * R, R), pl.ds(j * C, C)],
                wbuf.at[j, pl.ds(si * R, R), :],
                dma_sems.at[j, si],
            )

        for j in range(NC):
            for si in range(S):
                chunk_copy(j, si).start()

        x_f = x_ref[...]
        lab = labels_ref[0, :]
        col = lax.broadcasted_iota(jnp.int32, (T, C), 1)

        s_tot = None
        c_tot = None
        for j in range(NC):
            for si in range(S):
                chunk_copy(j, si).wait()
            logits = lax.dot_general(
                x_f, wbuf[j], (((1,), (0,)), ((), ())),
                preferred_element_type=jnp.float32,
                precision=lax.Precision.DEFAULT,
            )
            s_j = jnp.sum(jnp.exp(logits), axis=1)
            local_col = lab - my_pos * V - j * C
            hit = col == local_col[:, None]
            c_j = jnp.sum(jnp.where(hit, logits, 0.0), axis=1)
            s_tot = s_j if s_tot is None else s_tot + s_j
            c_tot = c_j if c_tot is None else c_tot + c_j

        sacc_ref[...] = jnp.stack([s_tot, c_tot])[None]

        pl.semaphore_wait(barrier_sem, N_DEV - 1)

        sends = []
        for d in range(1, N_DEV):
            peer = (my_pos + d) % N_DEV
            rdma = pltpu.make_async_remote_copy(
                src_ref=sacc_ref,
                dst_ref=comm_ref.at[pl.ds(my_pos, 1)],
                send_sem=send_sems.at[d],
                recv_sem=recv_sems.at[my_pos],
                device_id=(peer,),
                device_id_type=pl.DeviceIdType.MESH,
            )
            rdma.start()
            sends.append(rdma)

        for d in range(1, N_DEV):
            src_pos = (my_pos + d) % N_DEV
            recv = pltpu.make_async_remote_copy(
                src_ref=sacc_ref,
                dst_ref=comm_ref.at[pl.ds(src_pos, 1)],
                send_sem=send_sems.at[d],
                recv_sem=recv_sems.at[src_pos],
                device_id=(src_pos,),
                device_id_type=pl.DeviceIdType.MESH,
            )
            recv.wait_recv()
        for rdma in sends:
            rdma.wait_send()

        all_st = comm_ref[...]
        dev = lax.broadcasted_iota(jnp.int32, (N_DEV, 2, T), 0)
        all_st = jnp.where(dev == my_pos, sacc_ref[...], all_st)
        Ssum = jnp.sum(all_st[:, 0, :], axis=0)
        Lsum = jnp.sum(all_st[:, 1, :], axis=0)
        out_ref[0, :] = jnp.log(Ssum) - Lsum

    out = pl.pallas_call(
        body,
        out_shape=jax.ShapeDtypeStruct((1, T), jnp.float32),
        in_specs=[
            pl.BlockSpec(memory_space=pltpu.VMEM),
            pl.BlockSpec(memory_space=pltpu.MemorySpace.HBM),
            pl.BlockSpec(memory_space=pltpu.VMEM),
        ],
        out_specs=pl.BlockSpec(memory_space=pltpu.VMEM),
        scratch_shapes=[
            pltpu.VMEM((NC, D, C), jnp.float32),
            pltpu.VMEM((1, 2, T), jnp.float32),
            pltpu.VMEM((N_DEV, 2, T), jnp.float32),
            pltpu.SemaphoreType.DMA((NC, S)),
            pltpu.SemaphoreType.DMA((N_DEV,)),
            pltpu.SemaphoreType.DMA((N_DEV,)),
        ],
        compiler_params=pltpu.CompilerParams(
            collective_id=0,
            vmem_limit_bytes=56 * 1024 * 1024,
        ),
    )(x, W, labels.reshape(1, T))
    return out.reshape(T)
